# baseline (speedup 1.0000x reference)
import functools

import jax
import jax.numpy as jnp
from jax import lax
from jax.experimental import pallas as pl
from jax.experimental.pallas import tpu as pltpu
from jax.experimental.pallas import tpu_sc as plsc

B = 16384
D = 64
H1 = 128
NC, NS = 2, 16
NW = NC * NS
BPW = B // NW
NB = 4
HBLK = 256
NBLK16 = HBLK // 16

_sc_mesh = plsc.VectorSubcoreMesh(core_axis_name="c", subcore_axis_name="s")


def _gather_one_table(idx_v, tabT, out, ring, stage, sems, base):
    NGH = HBLK // NB

    def scal(j):
        return idx_v[pl.ds(j, 16)][0]

    def fire(h, g, slot):
        for t in range(NB):
            i = scal(h * HBLK + g * NB + t)
            off = pl.multiple_of((i >> 7) * 128, 128)
            pltpu.async_copy(
                tabT.at[:, pl.ds(off, 128)],
                ring.at[slot * NB + t], sems[slot])

    def wait(slot):
        for t in range(NB):
            pltpu.make_async_copy(
                tabT.at[:, pl.ds(0, 128)],
                ring.at[slot * NB + t], sems[slot]).wait()

    def extract(h, g, slot):
        for t in range(NB):
            i = scal(h * HBLK + g * NB + t)
            lane = i & 127
            kk = g * NB + t
            @pl.loop(0, D // 16)
            def _dq(dq):
                for dd in range(16):
                    d = dq * 16 + dd
                    vec = ring[slot * NB + t, d, pl.ds(lane, 16)]
                    stage[kk, pl.ds(d, 16)] = vec

    for h in range(BPW // HBLK):
        fire(h, 0, 0)

        @pl.loop(0, NGH // 2)
        def _pair(p):
            g0 = p * 2
            fire(h, g0 + 1, 1)
            wait(0)
            extract(h, g0, 0)

            @pl.when(p < NGH // 2 - 1)
            def _():
                fire(h, g0 + 2, 0)

            wait(1)
            extract(h, g0 + 1, 1)

        pltpu.sync_copy(stage, out.at[pl.ds(base + h * HBLK, HBLK)])


@functools.partial(
    pl.kernel,
    mesh=_sc_mesh,
    out_type=[
        jax.ShapeDtypeStruct((B, 128), jnp.float32),
        jax.ShapeDtypeStruct((B, 128), jnp.float32),
    ],
    scratch_types=[
        pltpu.VMEM((BPW + 16,), jnp.int32),
        pltpu.VMEM((BPW + 16,), jnp.int32),
        pltpu.VMEM((2 * NB, D, 128), jnp.float32),
        pltpu.VMEM((HBLK, 128), jnp.float32),
        pltpu.VMEM((16,), jnp.float32),
        pltpu.SemaphoreType.DMA,
        pltpu.SemaphoreType.DMA,
    ],
)
def _sc_gather(item_idx, user_idx, item_tabT, user_tabT,
               item_out, user_out, iidx_v, uidx_v, ring, stage, _pad,
               sem0, sem1):
    wid = lax.axis_index("s") * NC + lax.axis_index("c")
    base = wid * BPW
    pltpu.sync_copy(item_idx.at[pl.ds(base, BPW)], iidx_v.at[pl.ds(0, BPW)])
    pltpu.sync_copy(user_idx.at[pl.ds(base, BPW)], uidx_v.at[pl.ds(0, BPW)])
    sems = (sem0, sem1)
    _gather_one_table(iidx_v, item_tabT, item_out, ring, stage, sems, base)
    _gather_one_table(uidx_v, user_tabT, user_out, ring, stage, sems, base)


def _mlp_body(ie, ue, w1a, w1b, b1, w2, b2, w3, b3, out):
    h1 = jnp.dot(ie[:, :D], w1a[...], preferred_element_type=jnp.float32)
    h1 = h1 + jnp.dot(ue[:, :D], w1b[...], preferred_element_type=jnp.float32)
    h1 = jnp.maximum(h1 + b1[...], 0.0)
    h2 = jnp.dot(h1, w2[...], preferred_element_type=jnp.float32)
    h2 = jnp.maximum(h2 + b2[...], 0.0)
    logits = jnp.sum(h2 * w3[...], axis=1) + b3[0, 0]
    out[...] = (1.0 / (1.0 + jnp.exp(-logits)))[None, None, :]


BB = 4096
G = B // BB


def kernel(item_vec, user_vec, item_table, user_table, W1, b1, W2, b2, W3, b3):
    iv = item_vec.astype(jnp.int32)
    uv = user_vec.astype(jnp.int32)
    item_emb, user_emb = _sc_gather(iv, uv, item_table.T, user_table.T)
    out2 = pl.pallas_call(
        _mlp_body,
        grid=(G,),
        in_specs=[
            pl.BlockSpec((BB, 128), lambda i: (i, 0)),
            pl.BlockSpec((BB, 128), lambda i: (i, 0)),
            pl.BlockSpec((D, H1), lambda i: (0, 0)),
            pl.BlockSpec((D, H1), lambda i: (0, 0)),
            pl.BlockSpec((1, H1), lambda i: (0, 0)),
            pl.BlockSpec((H1, D), lambda i: (0, 0)),
            pl.BlockSpec((1, D), lambda i: (0, 0)),
            pl.BlockSpec((1, D), lambda i: (0, 0)),
            pl.BlockSpec((1, 1), lambda i: (0, 0)),
        ],
        out_specs=pl.BlockSpec((1, 1, BB), lambda i: (i, 0, 0)),
        out_shape=jax.ShapeDtypeStruct((G, 1, BB), jnp.float32),
    )(item_emb, user_emb, W1[:D], W1[D:], b1.reshape(1, H1),
      W2, b2.reshape(1, D), W3.reshape(1, D), b3.reshape(1, 1))
    return out2.reshape(-1)


# speedup vs baseline: 1.0114x; 1.0114x over previous
import functools

import jax
import jax.numpy as jnp
from jax import lax
from jax.experimental import pallas as pl
from jax.experimental.pallas import tpu as pltpu
from jax.experimental.pallas import tpu_sc as plsc

B = 16384
D = 64
H1 = 128
NC, NS = 2, 16
NW = NC * NS
BPW = B // NW
NB = 4
HBLK = 256
NBLK16 = HBLK // 16

_sc_mesh = plsc.VectorSubcoreMesh(core_axis_name="c", subcore_axis_name="s")


def _gather_one_table(idx_v, tabT, out, ring, stage, sems, base):
    NGH = HBLK // NB

    def scal(j):
        return idx_v[pl.ds(j, 16)][0]

    def fire(h, g, slot):
        for t in range(NB):
            i = scal(h * HBLK + g * NB + t)
            off = pl.multiple_of((i >> 7) * 128, 128)
            pltpu.async_copy(
                tabT.at[:, pl.ds(off, 128)],
                ring.at[slot * NB + t], sems[slot])

    def wait(slot):
        for t in range(NB):
            pltpu.make_async_copy(
                tabT.at[:, pl.ds(0, 128)],
                ring.at[slot * NB + t], sems[slot]).wait()

    def extract(h, g, slot):
        for t in range(NB):
            i = scal(h * HBLK + g * NB + t)
            lane = i & 127
            kk = g * NB + t
            @pl.loop(0, D // 16)
            def _dq(dq):
                for dd in range(16):
                    d = dq * 16 + dd
                    vec = ring[slot * NB + t, d, pl.ds(lane, 16)]
                    stage[kk, pl.ds(d, 16)] = vec

    for h in range(BPW // HBLK):
        fire(h, 0, 0)

        @pl.loop(0, NGH // 2)
        def _pair(p):
            g0 = p * 2
            fire(h, g0 + 1, 1)
            wait(0)
            extract(h, g0, 0)

            @pl.when(p < NGH // 2 - 1)
            def _():
                fire(h, g0 + 2, 0)

            wait(1)
            extract(h, g0 + 1, 1)

        pltpu.sync_copy(stage, out.at[pl.ds(base + h * HBLK, HBLK)])


@functools.partial(
    pl.kernel,
    mesh=_sc_mesh,
    out_type=[
        jax.ShapeDtypeStruct((B, 128), jnp.float32),
        jax.ShapeDtypeStruct((B, 128), jnp.float32),
    ],
    scratch_types=[
        pltpu.VMEM((BPW + 16,), jnp.int32),
        pltpu.VMEM((BPW + 16,), jnp.int32),
        pltpu.VMEM((2 * NB, D, 128), jnp.float32),
        pltpu.VMEM((HBLK, 128), jnp.float32),
        pltpu.VMEM((16,), jnp.float32),
        pltpu.SemaphoreType.DMA,
        pltpu.SemaphoreType.DMA,
    ],
)
def _sc_gather(item_idx, user_idx, item_tabT, user_tabT,
               item_out, user_out, iidx_v, uidx_v, ring, stage, _pad,
               sem0, sem1):
    wid = lax.axis_index("s") * NC + lax.axis_index("c")
    base = wid * BPW
    pltpu.sync_copy(item_idx.at[pl.ds(base, BPW)], iidx_v.at[pl.ds(0, BPW)])
    pltpu.sync_copy(user_idx.at[pl.ds(base, BPW)], uidx_v.at[pl.ds(0, BPW)])
    sems = (sem0, sem1)
    _gather_one_table(iidx_v, item_tabT, item_out, ring, stage, sems, base)
    _gather_one_table(uidx_v, user_tabT, user_out, ring, stage, sems, base)


def _mlp_body(ie, ue, w1a, w1b, b1, w2, b2, w3, b3, out):
    h1 = jnp.dot(ie[:, :D], w1a[...], preferred_element_type=jnp.float32)
    h1 = h1 + jnp.dot(ue[:, :D], w1b[...], preferred_element_type=jnp.float32)
    h1 = jnp.maximum(h1 + b1[...], 0.0)
    h2 = jnp.dot(h1, w2[...], preferred_element_type=jnp.float32)
    h2 = jnp.maximum(h2 + b2[...], 0.0)
    logits = jnp.sum(h2 * w3[...], axis=1) + b3[0, 0]
    out[...] = (1.0 / (1.0 + jnp.exp(-logits)))[None, None, :]


BB = 2048
G = B // BB


def kernel(item_vec, user_vec, item_table, user_table, W1, b1, W2, b2, W3, b3):
    iv = item_vec.astype(jnp.int32)
    uv = user_vec.astype(jnp.int32)
    item_emb, user_emb = _sc_gather(iv, uv, item_table.T, user_table.T)
    out2 = pl.pallas_call(
        _mlp_body,
        grid=(G,),
        in_specs=[
            pl.BlockSpec((BB, 128), lambda i: (i, 0)),
            pl.BlockSpec((BB, 128), lambda i: (i, 0)),
            pl.BlockSpec((D, H1), lambda i: (0, 0)),
            pl.BlockSpec((D, H1), lambda i: (0, 0)),
            pl.BlockSpec((1, H1), lambda i: (0, 0)),
            pl.BlockSpec((H1, D), lambda i: (0, 0)),
            pl.BlockSpec((1, D), lambda i: (0, 0)),
            pl.BlockSpec((1, D), lambda i: (0, 0)),
            pl.BlockSpec((1, 1), lambda i: (0, 0)),
        ],
        out_specs=pl.BlockSpec((1, 1, BB), lambda i: (i, 0, 0)),
        out_shape=jax.ShapeDtypeStruct((G, 1, BB), jnp.float32),
    )(item_emb, user_emb, W1[:D], W1[D:], b1.reshape(1, H1),
      W2, b2.reshape(1, D), W3.reshape(1, D), b3.reshape(1, 1))
    return out2.reshape(-1)
